# baseline (speedup 1.0000x reference)
import functools

import jax
import jax.numpy as jnp
from jax import lax
from jax.experimental import pallas as pl
from jax.experimental.pallas import tpu as pltpu
from jax.experimental.pallas import tpu_sc as plsc

F = 32
BATCH = 16384
NU = 1000000
NM = 100000
NC = 2
NS = 16
LANES = 16
FPC = F // NC
EPT = BATCH // NS
GROUPS = EPT // LANES

HALF = 499968
UMAIN = 2 * HALF
NTAIL = NU - UMAIN
SENT = 128
UBUF = UMAIN + SENT
SEG_A = 33280
SEG_B = 34048
NSLOT = 3 * FPC


def _mf_main(u_hbm, m_hbm, UT_hbm, MT_hbm, utail_hbm, part_hbm,
             usr_v, midx_v, uidx_v, uv0, uv1, mt0, mt1,
             acc_v, tbase_v, tmask_v, utail_v, zb_v,
             u_sh, m_sh0, m_sh1,
             stage_sem, mstage_sem, gather_sem):
    c = lax.axis_index("c")
    s = lax.axis_index("s")
    lanes = lax.iota(jnp.int32, LANES)
    uvs = (uv0, uv1)
    mts = (mt0, mt1)
    m_shs = (m_sh0, m_sh1)

    pltpu.sync_copy(u_hbm.at[pl.ds(s * EPT, EPT)], usr_v)
    pltpu.sync_copy(m_hbm.at[pl.ds(s * EPT, EPT)], midx_v)
    pltpu.sync_copy(utail_hbm, utail_v)

    @pl.when(s == NS - 1)
    def _zero_sentinels():
        zero = jnp.zeros((LANES,), jnp.float32)
        for g in range(SENT // LANES):
            zb_v[pl.ds(g * LANES, LANES)] = zero
        pltpu.sync_copy(zb_v, u_sh.at[pl.ds(UMAIN, SENT)])

    base_col = c * FPC

    def prep_body(g, ht):
        sl = pl.ds(g * LANES, LANES)
        u = usr_v[sl]
        sent = UMAIN + (g % 8) * LANES + lanes
        uidx_v[sl] = jnp.where(u < UMAIN, u, sent)
        acc_v[sl] = jnp.zeros((LANES,), jnp.float32)
        istail = u >= UMAIN
        tmask_v[sl] = jnp.where(istail, 1.0, 0.0).astype(jnp.float32)
        tbase_v[sl] = jnp.where(istail, (u - UMAIN) * F + base_col, 0)
        anyt = lax.reduce_max(
            jnp.where(istail, 1, 0).astype(jnp.int32), (0,))
        return jnp.maximum(ht, anyt)

    has_tail = lax.fori_loop(0, GROUPS, prep_body, jnp.int32(0))

    def issue_u_stage(f, h):
        d = c * FPC + f

        @pl.when(s < NS - 2)
        def _():
            off = pl.multiple_of(h * HALF + s * SEG_A, 128)
            pltpu.async_copy(
                UT_hbm.at[d, pl.ds(off, SEG_A)],
                u_sh.at[pl.ds(off, SEG_A)], stage_sem)

        @pl.when(s == NS - 2)
        def _():
            off = h * HALF + 14 * SEG_A
            pltpu.async_copy(
                UT_hbm.at[d, pl.ds(off, SEG_B)],
                u_sh.at[pl.ds(off, SEG_B)], stage_sem)

    def drain_u_stage():
        @pl.when(s < NS - 2)
        def _():
            pltpu.make_async_copy(
                UT_hbm.at[0, pl.ds(0, SEG_A)],
                u_sh.at[pl.ds(0, SEG_A)], stage_sem).wait()

        @pl.when(s == NS - 2)
        def _():
            pltpu.make_async_copy(
                UT_hbm.at[0, pl.ds(0, SEG_B)],
                u_sh.at[pl.ds(0, SEG_B)], stage_sem).wait()

    def issue_m_stage(f):
        @pl.when(s == NS - 1)
        def _():
            pltpu.async_copy(
                MT_hbm.at[c * FPC + f], m_shs[f % 2], mstage_sem)

    def drain_m_stage():
        @pl.when(s == NS - 1)
        def _():
            pltpu.make_async_copy(MT_hbm.at[0], m_sh0, mstage_sem).wait()

    def fma(f):
        uv = uvs[f % 2]
        mt = mts[f % 2]

        @pl.when(has_tail == 0)
        def _plain():
            def body(g, carry):
                sl = pl.ds(g * LANES, LANES)
                acc_v[sl] = acc_v[sl] + uv[sl] * mt[sl]
                return carry

            lax.fori_loop(0, GROUPS, body, 0)

        @pl.when(has_tail == 1)
        def _with_tail():
            def body(g, carry, f=f):
                sl = pl.ds(g * LANES, LANES)
                tv = plsc.load_gather(utail_v, [tbase_v[sl] + f])
                uval = uv[sl] + tv * tmask_v[sl]
                acc_v[sl] = acc_v[sl] + uval * mt[sl]
                return carry

            lax.fori_loop(0, GROUPS, body, 0)

    issue_m_stage(0)
    inflight = None
    for j in range(NSLOT):
        f, sub = divmod(j, 3)
        if inflight is not None:
            with jax.named_scope("gwait"):
                inflight.wait()
            inflight = None
        if sub == 0:
            with jax.named_scope("mdrain"):
                drain_m_stage()
            with jax.named_scope("bar"):
                plsc.subcore_barrier()
        elif sub == 2:
            with jax.named_scope("udrain"):
                drain_u_stage()
                drain_u_stage()
            with jax.named_scope("bar"):
                plsc.subcore_barrier()
        if sub == 0:
            with jax.named_scope("issue"):
                if f + 1 < FPC:
                    issue_m_stage(f + 1)
                issue_u_stage(f, 0)
                issue_u_stage(f, 1)
                inflight = pltpu.async_copy(
                    m_shs[f % 2].at[midx_v], mts[f % 2], gather_sem)
        elif sub == 2:
            with jax.named_scope("fire"):
                inflight = pltpu.async_copy(
                    u_sh.at[uidx_v], uvs[f % 2], gather_sem)
        if j >= 4 and (j - 4) % 3 == 0:
            with jax.named_scope("fma"):
                fma((j - 4) // 3)
    inflight.wait()
    fma(FPC - 1)

    pltpu.sync_copy(acc_v, part_hbm.at[c, pl.ds(s * EPT, EPT)])


def _add_kernel(part_hbm, out_hbm, a_v, b_v):
    wid = lax.axis_index("s") * NC + lax.axis_index("c")
    n = BATCH // (NC * NS)
    base = wid * n
    pltpu.sync_copy(part_hbm.at[0, pl.ds(base, n)], a_v)
    pltpu.sync_copy(part_hbm.at[1, pl.ds(base, n)], b_v)

    def body(g, carry):
        sl = pl.ds(g * LANES, LANES)
        a_v[sl] = a_v[sl] + b_v[sl]
        return carry

    lax.fori_loop(0, n // LANES, body, 0)
    pltpu.sync_copy(a_v, out_hbm.at[pl.ds(base, n)])


def kernel(users, movies, U, M):
    users = users.astype(jnp.int32)
    movies = movies.astype(jnp.int32)
    UT = U.T
    MT = M.T
    utail = U[UMAIN:].reshape(-1)

    mesh = plsc.VectorSubcoreMesh(core_axis_name="c", subcore_axis_name="s")
    params = pltpu.CompilerParams(needs_layout_passes=False)

    k1 = functools.partial(
        pl.kernel,
        mesh=mesh,
        compiler_params=params,
        out_type=jax.ShapeDtypeStruct((NC, BATCH), jnp.float32),
        scratch_types=[
            pltpu.VMEM((EPT,), jnp.int32),
            pltpu.VMEM((EPT,), jnp.int32),
            pltpu.VMEM((EPT,), jnp.int32),
            pltpu.VMEM((EPT,), jnp.float32),
            pltpu.VMEM((EPT,), jnp.float32),
            pltpu.VMEM((EPT,), jnp.float32),
            pltpu.VMEM((EPT,), jnp.float32),
            pltpu.VMEM((EPT,), jnp.float32),
            pltpu.VMEM((EPT,), jnp.int32),
            pltpu.VMEM((EPT,), jnp.float32),
            pltpu.VMEM((NTAIL * F,), jnp.float32),
            pltpu.VMEM((SENT,), jnp.float32),
            pltpu.VMEM_SHARED((UBUF,), jnp.float32),
            pltpu.VMEM_SHARED((NM,), jnp.float32),
            pltpu.VMEM_SHARED((NM,), jnp.float32),
            pltpu.SemaphoreType.DMA,
            pltpu.SemaphoreType.DMA,
            pltpu.SemaphoreType.DMA,
        ],
    )(_mf_main)
    partials = k1(users, movies, UT, MT, utail)

    k2 = functools.partial(
        pl.kernel,
        mesh=mesh,
        compiler_params=params,
        out_type=jax.ShapeDtypeStruct((BATCH,), jnp.float32),
        scratch_types=[
            pltpu.VMEM((BATCH // (NC * NS),), jnp.float32),
            pltpu.VMEM((BATCH // (NC * NS),), jnp.float32),
        ],
    )(_add_kernel)
    return k2(partials)


# speedup vs baseline: 1.0100x; 1.0100x over previous
import functools

import jax
import jax.numpy as jnp
from jax import lax
from jax.experimental import pallas as pl
from jax.experimental.pallas import tpu as pltpu
from jax.experimental.pallas import tpu_sc as plsc

F = 32
BATCH = 16384
NU = 1000000
NM = 100000
NC = 2
NS = 16
LANES = 16
FPC = F // NC
EPT = BATCH // NS
GROUPS = EPT // LANES

HALF = 499968
UMAIN = 2 * HALF
NTAIL = NU - UMAIN
SENT = 128
UBUF = UMAIN + SENT
SEG_A = 33280
SEG_B = 34048
NSLOT = 3 * FPC


def _mf_main(u_hbm, m_hbm, UT_hbm, MT_hbm, utail_hbm, part_hbm,
             usr_v, midx_v, uidx_v, uv0, uv1, mt0, mt1,
             acc_v, tbase_v, tmask_v, utail_v, zb_v,
             u_sh, m_sh0, m_sh1,
             stage_sem, mstage_sem, gather_sem):
    c = lax.axis_index("c")
    s = lax.axis_index("s")
    lanes = lax.iota(jnp.int32, LANES)
    uvs = (uv0, uv1)
    mts = (mt0, mt1)
    m_shs = (m_sh0, m_sh1)

    pltpu.sync_copy(u_hbm.at[pl.ds(s * EPT, EPT)], usr_v)
    pltpu.sync_copy(m_hbm.at[pl.ds(s * EPT, EPT)], midx_v)
    pltpu.sync_copy(utail_hbm, utail_v)

    @pl.when(s == NS - 1)
    def _zero_sentinels():
        zero = jnp.zeros((LANES,), jnp.float32)
        for g in range(SENT // LANES):
            zb_v[pl.ds(g * LANES, LANES)] = zero
        pltpu.sync_copy(zb_v, u_sh.at[pl.ds(UMAIN, SENT)])

    base_col = c * FPC

    def prep_body(g, ht):
        sl = pl.ds(g * LANES, LANES)
        u = usr_v[sl]
        sent = UMAIN + (g % 8) * LANES + lanes
        uidx_v[sl] = jnp.where(u < UMAIN, u, sent)
        acc_v[sl] = jnp.zeros((LANES,), jnp.float32)
        istail = u >= UMAIN
        tmask_v[sl] = jnp.where(istail, 1.0, 0.0).astype(jnp.float32)
        tbase_v[sl] = jnp.where(istail, (u - UMAIN) * F + base_col, 0)
        anyt = lax.reduce_max(
            jnp.where(istail, 1, 0).astype(jnp.int32), (0,))
        return jnp.maximum(ht, anyt)

    has_tail = lax.fori_loop(0, GROUPS, prep_body, jnp.int32(0))

    def issue_u_stage(f, h):
        d = c * FPC + f

        @pl.when(s < NS - 2)
        def _():
            off = pl.multiple_of(h * HALF + s * SEG_A, 128)
            pltpu.async_copy(
                UT_hbm.at[d, pl.ds(off, SEG_A)],
                u_sh.at[pl.ds(off, SEG_A)], stage_sem)

        @pl.when(s == NS - 2)
        def _():
            off = h * HALF + 14 * SEG_A
            pltpu.async_copy(
                UT_hbm.at[d, pl.ds(off, SEG_B)],
                u_sh.at[pl.ds(off, SEG_B)], stage_sem)

    def drain_u_stage():
        @pl.when(s < NS - 2)
        def _():
            pltpu.make_async_copy(
                UT_hbm.at[0, pl.ds(0, SEG_A)],
                u_sh.at[pl.ds(0, SEG_A)], stage_sem).wait()

        @pl.when(s == NS - 2)
        def _():
            pltpu.make_async_copy(
                UT_hbm.at[0, pl.ds(0, SEG_B)],
                u_sh.at[pl.ds(0, SEG_B)], stage_sem).wait()

    def issue_m_stage(f):
        @pl.when(s == NS - 1)
        def _():
            pltpu.async_copy(
                MT_hbm.at[c * FPC + f], m_shs[f % 2], mstage_sem)

    def drain_m_stage():
        @pl.when(s == NS - 1)
        def _():
            pltpu.make_async_copy(MT_hbm.at[0], m_sh0, mstage_sem).wait()

    def fma(f):
        uv = uvs[f % 2]
        mt = mts[f % 2]

        @pl.when(has_tail == 0)
        def _plain():
            def body(g, carry):
                sl = pl.ds(g * LANES, LANES)
                acc_v[sl] = acc_v[sl] + uv[sl] * mt[sl]
                return carry

            lax.fori_loop(0, GROUPS, body, 0)

        @pl.when(has_tail == 1)
        def _with_tail():
            def body(g, carry, f=f):
                sl = pl.ds(g * LANES, LANES)
                tv = plsc.load_gather(utail_v, [tbase_v[sl] + f])
                uval = uv[sl] + tv * tmask_v[sl]
                acc_v[sl] = acc_v[sl] + uval * mt[sl]
                return carry

            lax.fori_loop(0, GROUPS, body, 0)

    issue_m_stage(0)
    ug = None
    mg = None
    for f in range(FPC):
        if ug is not None:
            ug.wait()
            ug = None
        drain_m_stage()
        plsc.subcore_barrier()
        if f + 1 < FPC:
            issue_m_stage(f + 1)
        issue_u_stage(f, 0)
        issue_u_stage(f, 1)
        mg = pltpu.async_copy(
            m_shs[f % 2].at[midx_v], mts[f % 2], gather_sem)
        if f >= 1:
            fma(f - 1)
        mg.wait()
        drain_u_stage()
        drain_u_stage()
        plsc.subcore_barrier()
        ug = pltpu.async_copy(u_sh.at[uidx_v], uvs[f % 2], gather_sem)
    ug.wait()
    fma(FPC - 1)

    pltpu.sync_copy(acc_v, part_hbm.at[c, pl.ds(s * EPT, EPT)])


def _add_kernel(part_hbm, out_hbm, a_v, b_v):
    wid = lax.axis_index("s") * NC + lax.axis_index("c")
    n = BATCH // (NC * NS)
    base = wid * n
    pltpu.sync_copy(part_hbm.at[0, pl.ds(base, n)], a_v)
    pltpu.sync_copy(part_hbm.at[1, pl.ds(base, n)], b_v)

    def body(g, carry):
        sl = pl.ds(g * LANES, LANES)
        a_v[sl] = a_v[sl] + b_v[sl]
        return carry

    lax.fori_loop(0, n // LANES, body, 0)
    pltpu.sync_copy(a_v, out_hbm.at[pl.ds(base, n)])


def kernel(users, movies, U, M):
    users = users.astype(jnp.int32)
    movies = movies.astype(jnp.int32)
    UT = U.T
    MT = M.T
    utail = U[UMAIN:].reshape(-1)

    mesh = plsc.VectorSubcoreMesh(core_axis_name="c", subcore_axis_name="s")
    params = pltpu.CompilerParams(needs_layout_passes=False)

    k1 = functools.partial(
        pl.kernel,
        mesh=mesh,
        compiler_params=params,
        out_type=jax.ShapeDtypeStruct((NC, BATCH), jnp.float32),
        scratch_types=[
            pltpu.VMEM((EPT,), jnp.int32),
            pltpu.VMEM((EPT,), jnp.int32),
            pltpu.VMEM((EPT,), jnp.int32),
            pltpu.VMEM((EPT,), jnp.float32),
            pltpu.VMEM((EPT,), jnp.float32),
            pltpu.VMEM((EPT,), jnp.float32),
            pltpu.VMEM((EPT,), jnp.float32),
            pltpu.VMEM((EPT,), jnp.float32),
            pltpu.VMEM((EPT,), jnp.int32),
            pltpu.VMEM((EPT,), jnp.float32),
            pltpu.VMEM((NTAIL * F,), jnp.float32),
            pltpu.VMEM((SENT,), jnp.float32),
            pltpu.VMEM_SHARED((UBUF,), jnp.float32),
            pltpu.VMEM_SHARED((NM,), jnp.float32),
            pltpu.VMEM_SHARED((NM,), jnp.float32),
            pltpu.SemaphoreType.DMA,
            pltpu.SemaphoreType.DMA,
            pltpu.SemaphoreType.DMA,
        ],
    )(_mf_main)
    partials = k1(users, movies, UT, MT, utail)

    k2 = functools.partial(
        pl.kernel,
        mesh=mesh,
        compiler_params=params,
        out_type=jax.ShapeDtypeStruct((BATCH,), jnp.float32),
        scratch_types=[
            pltpu.VMEM((BATCH // (NC * NS),), jnp.float32),
            pltpu.VMEM((BATCH // (NC * NS),), jnp.float32),
        ],
    )(_add_kernel)
    return k2(partials)
